# baseline (speedup 1.0000x reference)
import functools

import jax
import jax.numpy as jnp
from jax import lax
from jax.experimental import pallas as pl
from jax.experimental.pallas import tpu as pltpu
from jax.experimental.pallas import tpu_sc as plsc

E = 320000
D = 128
PD = 64
N = 10000

SEGS = (192000, 128000)
SEG_OFF = (0, 192000)
NC = 2
NS = 16
NW = NC * NS
CHUNK = 40
NBUF = 5
RPS = 624
TAIL = N - NS * RPS


def _gate_body(lp_ref, x_ref, wt_ref, out_ref):
    x = x_ref[...]
    a = x[:, :PD]
    b = x[:, PD:]
    wt = wt_ref[...]
    ga = jnp.clip(lax.dot(a, wt, preferred_element_type=jnp.float32), 0.0, 1.0)
    gb = jnp.clip(lax.dot(b, wt, preferred_element_type=jnp.float32), 0.0, 1.0)
    at = jnp.concatenate([a, a], axis=1)
    bt = jnp.concatenate([b, b], axis=1)
    lp = jnp.abs(lp_ref[0])
    out_ref[...] = lp * (at * ga + bt * gb) + x


def _gate(x, wt, lp, seg):
    eseg = SEGS[seg]
    gb = 16000
    nblk = eseg // gb
    blk0 = SEG_OFF[seg] // gb
    return pl.pallas_call(
        _gate_body,
        grid=(nblk,),
        in_specs=[
            pl.BlockSpec(memory_space=pltpu.SMEM),
            pl.BlockSpec((gb, D), lambda i, _o=blk0: (i + _o, 0)),
            pl.BlockSpec((PD, D), lambda i: (0, 0)),
        ],
        out_specs=pl.BlockSpec((gb, D), lambda i: (i, 0)),
        out_shape=jax.ShapeDtypeStruct((eseg, D), jnp.float32),
    )(lp, x, wt)


def _sc_scatter(comb, idx, init, seg):
    mesh = plsc.VectorSubcoreMesh(core_axis_name="c", subcore_axis_name="s")
    ibase0 = SEG_OFF[seg]
    epw = SEGS[seg] // NW
    nchunk = epw // CHUNK

    @functools.partial(
        pl.kernel,
        mesh=mesh,
        out_type=jax.ShapeDtypeStruct((NC * N, D), jnp.float32),
        scratch_types=(
            [pltpu.VMEM((CHUNK, D), jnp.float32) for _ in range(NBUF)]
            + [pltpu.VMEM((CHUNK,), jnp.int32) for _ in range(NBUF)]
            + [pltpu.VMEM_SHARED((N, D), jnp.float32)]
            + [pltpu.SemaphoreType.DMA for _ in range(NBUF)]
        ),
    )
    def run(comb_hbm, idx_hbm, init_hbm, out_hbm,
            eb0, eb1, eb2, eb3, eb4, ib0, ib1, ib2, ib3, ib4,
            acc, sg0, sg1, sg2, sg3, sg4):
        ebufs = [eb0, eb1, eb2, eb3, eb4]
        ibufs = [ib0, ib1, ib2, ib3, ib4]
        sgs = [sg0, sg1, sg2, sg3, sg4]
        c = lax.axis_index("c")
        s = lax.axis_index("s")
        wid = s * NC + c
        base = wid * epw
        ibase = ibase0 + base

        for b in range(NBUF):
            off = b * CHUNK
            pltpu.async_copy(comb_hbm.at[pl.ds(base + off, CHUNK)],
                             ebufs[b], sgs[b])
            pltpu.async_copy(idx_hbm.at[pl.ds(ibase + off, CHUNK)],
                             ibufs[b], sgs[b])

        pltpu.sync_copy(init_hbm.at[pl.ds(c * N + s * RPS, RPS)],
                        acc.at[pl.ds(s * RPS, RPS)])

        @pl.when(s == 0)
        def _():
            pltpu.sync_copy(init_hbm.at[pl.ds(c * N + NS * RPS, TAIL)],
                            acc.at[pl.ds(NS * RPS, TAIL)])

        plsc.subcore_barrier()

        def body(g, carry):
            for b in range(NBUF):
                i = g * NBUF + b
                pltpu.make_async_copy(
                    comb_hbm.at[pl.ds(base, CHUNK)], ebufs[b], sgs[b]).wait()
                pltpu.make_async_copy(
                    idx_hbm.at[pl.ds(base, CHUNK)], ibufs[b], sgs[b]).wait()
                pltpu.sync_copy(ebufs[b], acc.at[ibufs[b]], add=True)
                nxt = i + NBUF

                @pl.when(nxt < nchunk)
                def _():
                    off = nxt * CHUNK
                    pltpu.async_copy(comb_hbm.at[pl.ds(base + off, CHUNK)],
                                     ebufs[b], sgs[b])
                    pltpu.async_copy(idx_hbm.at[pl.ds(ibase + off, CHUNK)],
                                     ibufs[b], sgs[b])
            return carry

        lax.fori_loop(0, nchunk // NBUF, body, 0)
        plsc.subcore_barrier()
        pltpu.sync_copy(acc.at[pl.ds(s * RPS, RPS)],
                        out_hbm.at[pl.ds(c * N + s * RPS, RPS)])

        @pl.when(s == 0)
        def _():
            pltpu.sync_copy(acc.at[pl.ds(NS * RPS, TAIL)],
                            out_hbm.at[pl.ds(c * N + NS * RPS, TAIL)])

    return run(comb, idx, init)


def _add_body(p_ref, q_ref, o_ref):
    o_ref[...] = p_ref[...] + q_ref[...]


def _final_add(partials):
    bn = 2000
    nblk = N // bn
    return pl.pallas_call(
        _add_body,
        grid=(nblk,),
        in_specs=[
            pl.BlockSpec((bn, D), lambda i: (i, 0)),
            pl.BlockSpec((bn, D), lambda i, _o=nblk: (i + _o, 0)),
        ],
        out_specs=pl.BlockSpec((bn, D), lambda i: (i, 0)),
        out_shape=jax.ShapeDtypeStruct((N, D), jnp.float32),
    )(partials, partials)


def kernel(x, index, dim, dim_size, W, learnable_param):
    del dim, dim_size
    wt = W.T
    idx = index.astype(jnp.int32)
    running = jnp.zeros((NC * N, D), jnp.float32)
    for k in range(len(SEGS)):
        comb_k = _gate(x, wt, learnable_param, k)
        running = _sc_scatter(comb_k, idx, running, k)
    return _final_add(running)


# speedup vs baseline: 1.0215x; 1.0215x over previous
import functools

import jax
import jax.numpy as jnp
from jax import lax
from jax.experimental import pallas as pl
from jax.experimental.pallas import tpu as pltpu
from jax.experimental.pallas import tpu_sc as plsc

E = 320000
D = 128
PD = 64
N = 10000

SEGS = (160000, 160000)
SEG_OFF = (0, 160000)
NC = 2
NS = 16
NW = NC * NS
CHUNK = 40
NBUF = 5
RPS = 624
TAIL = N - NS * RPS


def _gate_body(lp_ref, x_ref, wt_ref, out_ref):
    x = x_ref[...]
    a = x[:, :PD]
    b = x[:, PD:]
    wt = wt_ref[...]
    ga = jnp.clip(lax.dot(a, wt, preferred_element_type=jnp.float32), 0.0, 1.0)
    gb = jnp.clip(lax.dot(b, wt, preferred_element_type=jnp.float32), 0.0, 1.0)
    at = jnp.concatenate([a, a], axis=1)
    bt = jnp.concatenate([b, b], axis=1)
    lp = jnp.abs(lp_ref[0])
    out_ref[...] = lp * (at * ga + bt * gb) + x


def _gate(x, wt, lp, seg):
    eseg = SEGS[seg]
    gb = 20000
    nblk = eseg // gb
    blk0 = SEG_OFF[seg] // gb
    return pl.pallas_call(
        _gate_body,
        grid=(nblk,),
        in_specs=[
            pl.BlockSpec(memory_space=pltpu.SMEM),
            pl.BlockSpec((gb, D), lambda i, _o=blk0: (i + _o, 0)),
            pl.BlockSpec((PD, D), lambda i: (0, 0)),
        ],
        out_specs=pl.BlockSpec((gb, D), lambda i: (i, 0)),
        out_shape=jax.ShapeDtypeStruct((eseg, D), jnp.float32),
    )(lp, x, wt)


def _sc_scatter(comb, idx, init, seg):
    mesh = plsc.VectorSubcoreMesh(core_axis_name="c", subcore_axis_name="s")
    ibase0 = SEG_OFF[seg]
    epw = SEGS[seg] // NW
    nchunk = epw // CHUNK

    @functools.partial(
        pl.kernel,
        mesh=mesh,
        out_type=jax.ShapeDtypeStruct((NC * N, D), jnp.float32),
        scratch_types=(
            [pltpu.VMEM((CHUNK, D), jnp.float32) for _ in range(NBUF)]
            + [pltpu.VMEM((CHUNK,), jnp.int32) for _ in range(NBUF)]
            + [pltpu.VMEM_SHARED((N, D), jnp.float32)]
            + [pltpu.SemaphoreType.DMA for _ in range(NBUF)]
        ),
    )
    def run(comb_hbm, idx_hbm, init_hbm, out_hbm,
            eb0, eb1, eb2, eb3, eb4, ib0, ib1, ib2, ib3, ib4,
            acc, sg0, sg1, sg2, sg3, sg4):
        ebufs = [eb0, eb1, eb2, eb3, eb4]
        ibufs = [ib0, ib1, ib2, ib3, ib4]
        sgs = [sg0, sg1, sg2, sg3, sg4]
        c = lax.axis_index("c")
        s = lax.axis_index("s")
        wid = s * NC + c
        base = wid * epw
        ibase = ibase0 + base

        for b in range(NBUF):
            off = b * CHUNK
            pltpu.async_copy(comb_hbm.at[pl.ds(base + off, CHUNK)],
                             ebufs[b], sgs[b])
            pltpu.async_copy(idx_hbm.at[pl.ds(ibase + off, CHUNK)],
                             ibufs[b], sgs[b])

        pltpu.sync_copy(init_hbm.at[pl.ds(c * N + s * RPS, RPS)],
                        acc.at[pl.ds(s * RPS, RPS)])

        @pl.when(s == 0)
        def _():
            pltpu.sync_copy(init_hbm.at[pl.ds(c * N + NS * RPS, TAIL)],
                            acc.at[pl.ds(NS * RPS, TAIL)])

        plsc.subcore_barrier()

        def body(g, carry):
            for b in range(NBUF):
                i = g * NBUF + b
                pltpu.make_async_copy(
                    comb_hbm.at[pl.ds(base, CHUNK)], ebufs[b], sgs[b]).wait()
                pltpu.make_async_copy(
                    idx_hbm.at[pl.ds(base, CHUNK)], ibufs[b], sgs[b]).wait()
                pltpu.sync_copy(ebufs[b], acc.at[ibufs[b]], add=True)
                nxt = i + NBUF

                @pl.when(nxt < nchunk)
                def _():
                    off = nxt * CHUNK
                    pltpu.async_copy(comb_hbm.at[pl.ds(base + off, CHUNK)],
                                     ebufs[b], sgs[b])
                    pltpu.async_copy(idx_hbm.at[pl.ds(ibase + off, CHUNK)],
                                     ibufs[b], sgs[b])
            return carry

        lax.fori_loop(0, nchunk // NBUF, body, 0)
        plsc.subcore_barrier()
        pltpu.sync_copy(acc.at[pl.ds(s * RPS, RPS)],
                        out_hbm.at[pl.ds(c * N + s * RPS, RPS)])

        @pl.when(s == 0)
        def _():
            pltpu.sync_copy(acc.at[pl.ds(NS * RPS, TAIL)],
                            out_hbm.at[pl.ds(c * N + NS * RPS, TAIL)])

    return run(comb, idx, init)


def _add_body(p_ref, q_ref, o_ref):
    o_ref[...] = p_ref[...] + q_ref[...]


def _final_add(partials):
    bn = 2000
    nblk = N // bn
    return pl.pallas_call(
        _add_body,
        grid=(nblk,),
        in_specs=[
            pl.BlockSpec((bn, D), lambda i: (i, 0)),
            pl.BlockSpec((bn, D), lambda i, _o=nblk: (i + _o, 0)),
        ],
        out_specs=pl.BlockSpec((bn, D), lambda i: (i, 0)),
        out_shape=jax.ShapeDtypeStruct((N, D), jnp.float32),
    )(partials, partials)


def kernel(x, index, dim, dim_size, W, learnable_param):
    del dim, dim_size
    wt = W.T
    idx = index.astype(jnp.int32)
    running = jnp.zeros((NC * N, D), jnp.float32)
    for k in range(len(SEGS)):
        comb_k = _gate(x, wt, learnable_param, k)
        running = _sc_scatter(comb_k, idx, running, k)
    return _final_add(running)
